# baseline (speedup 1.0000x reference)
import functools

import jax
import jax.numpy as jnp
from jax import lax
from jax.experimental import pallas as pl
from jax.experimental.pallas import tpu as pltpu
from jax.experimental.pallas import tpu_sc as plsc

_NUM_CORES = 2
_NUM_SUBCORES = 16
_NW = _NUM_CORES * _NUM_SUBCORES
_GATHER_CHUNK = 128


@functools.partial(jax.jit, static_argnames=("b_per_w", "t_len", "d"))
def _embed_lookup(W_emb, x_flat, W_pos, *, b_per_w, t_len, d):
    pos_half = b_per_w // 2

    mesh = plsc.VectorSubcoreMesh(core_axis_name="c", subcore_axis_name="s")

    @functools.partial(
        pl.kernel,
        out_type=jax.ShapeDtypeStruct((x_flat.shape[0], d), jnp.float32),
        mesh=mesh,
        scratch_types=[
            pltpu.VMEM((b_per_w,), jnp.int32),
            pltpu.VMEM((b_per_w, d), jnp.float32),
            pltpu.VMEM((pos_half, d), jnp.float32),
            pltpu.SemaphoreType.DMA,
        ],
        compiler_params=pltpu.CompilerParams(use_tc_tiling_on_sc=False),
    )
    def k(emb_hbm, idx_hbm, pos_hbm, out_hbm, idx_v, buf, pos_v, sem):
        wid = lax.axis_index("s") * _NUM_CORES + lax.axis_index("c")
        base = wid * b_per_w
        pos_off = (base * 1) % t_len

        pltpu.sync_copy(idx_hbm.at[pl.ds(base, b_per_w)], idx_v)

        copies = []
        for c in range(b_per_w // _GATHER_CHUNK):
            copies.append(
                pltpu.make_async_copy(
                    emb_hbm.at[idx_v.at[pl.ds(c * _GATHER_CHUNK, _GATHER_CHUNK)]],
                    buf.at[pl.ds(c * _GATHER_CHUNK, _GATHER_CHUNK)],
                    sem,
                )
            )
        for c in copies:
            c.start()
        for c in copies:
            c.wait()

        for h in range(2):
            pltpu.sync_copy(
                pos_hbm.at[pl.ds(pos_off + h * pos_half, pos_half)], pos_v
            )

            def add_row(r, _, h=h):
                for j in range(d // 16):
                    sl = pl.ds(j * 16, 16)
                    buf[h * pos_half + r, sl] = (
                        buf[h * pos_half + r, sl] + pos_v[r, sl]
                    )
                return 0

            lax.fori_loop(0, pos_half, add_row, 0)

        pltpu.sync_copy(buf, out_hbm.at[pl.ds(base, b_per_w)])

    return k(W_emb, x_flat, W_pos)


def kernel(x, W_emb, W_pos):
    bsz, t_len = x.shape
    d = W_emb.shape[1]
    x_flat = x.reshape(-1).astype(jnp.int32)
    b_per_w = x_flat.shape[0] // _NW
    out = _embed_lookup(W_emb, x_flat, W_pos, b_per_w=b_per_w, t_len=t_len, d=d)
    return out.reshape(bsz, t_len, d)
